# baseline (speedup 1.0000x reference)
import functools

import jax
import jax.numpy as jnp
from jax import lax
from jax.experimental import pallas as pl
from jax.experimental.pallas import tpu as pltpu
from jax.experimental.pallas import tpu_sc as plsc

N = 100000
BATCH = 4
MAX_SEG = 400
NSEG = BATCH * MAX_SEG
HIDDEN = 256
NQ = 4
QPTS = N // NQ
CH = 48
NVEC = CH // 16
NCHUNK = QPTS // CH
TAILP = QPTS - NCHUNK * CH
NUNIT = 8

_f32 = jnp.float32


def _make_sc_scatter():
    mesh = plsc.VectorSubcoreMesh(core_axis_name="c", subcore_axis_name="s")
    out_type = [
        jax.ShapeDtypeStruct((NUNIT * NQ * NSEG * 64,), _f32),
        jax.ShapeDtypeStruct((NQ * NSEG,), _f32),
    ]
    scratch_types = [
        pltpu.VMEM((NSEG * 64,), _f32),
        pltpu.VMEM((NSEG,), _f32),
        pltpu.VMEM((CH,), jnp.int32),
        pltpu.VMEM((CH,), jnp.int32),
        pltpu.VMEM((CH,), jnp.int32),
        pltpu.VMEM((CH, 64), _f32),
        pltpu.VMEM((CH, 64), _f32),
        pltpu.VMEM((CH, 128), _f32),
        pltpu.VMEM((CH, 128), _f32),
        pltpu.SemaphoreType.DMA,
        pltpu.SemaphoreType.DMA,
        pltpu.SemaphoreType.DMA,
    ]

    @functools.partial(
        pl.kernel, mesh=mesh, out_type=out_type, scratch_types=scratch_types,
        compiler_params=pltpu.CompilerParams(needs_layout_passes=False))
    def k(bat_hbm, seg_hbm, f0_hbm, f1_hbm, f2_hbm, f3_hbm,
          out_hbm, cnt_hbm,
          acc, cnt, btv, psv, idxv, datA0, datA1, datB0, datB1,
          sem_i, sem_d0, sem_d1):
        cc = lax.axis_index("c")
        ss = lax.axis_index("s")
        wid = ss * 2 + cc
        uu = wid % NUNIT
        qq = wid // NUNIT
        qbase = qq * QPTS
        is_wide = uu >= 2
        coff_s = ((uu % 2) == 1).astype(jnp.int32) * 64

        def _z(i, carry):
            for kk in range(4):
                acc[pl.ds(i * 64 + kk * 16, 16)] = jnp.zeros((16,), _f32)
            return carry
        lax.fori_loop(0, NSEG, _z, 0)

        def _zc(i, carry):
            cnt[pl.ds(i * 16, 16)] = jnp.zeros((16,), _f32)
            return carry
        lax.fori_loop(0, NSEG // 16, _zc, 0)

        ones16 = jnp.ones((16,), _f32)
        tail_mask = jax.lax.iota(jnp.int32, 16) >= 8

        def _issue_idx(start, npts):
            sl = pl.ds(start, npts)
            dst = pl.ds(0, npts)
            pltpu.async_copy(bat_hbm.at[sl], btv.at[dst], sem_i)
            pltpu.async_copy(seg_hbm.at[sl], psv.at[dst], sem_i)

        def _wait_idx(npts):
            sl = pl.ds(0, npts)
            pltpu.make_async_copy(bat_hbm.at[sl], btv.at[sl], sem_i).wait()
            pltpu.make_async_copy(seg_hbm.at[sl], psv.at[sl], sem_i).wait()

        def _issue_dat(start, npts, datA, datB, sem_d):
            sl = pl.ds(start, npts)
            dstA = datA.at[pl.ds(0, npts)]
            dstB = datB.at[pl.ds(0, npts)]

            @pl.when(uu == 0)
            def _():
                pltpu.async_copy(f0_hbm.at[sl], dstA, sem_d)

            @pl.when(uu == 1)
            def _():
                pltpu.async_copy(f1_hbm.at[sl], dstA, sem_d)

            @pl.when((uu == 2) | (uu == 3))
            def _():
                pltpu.async_copy(f2_hbm.at[sl, pl.ds(0, 128)], dstB, sem_d)

            @pl.when((uu == 4) | (uu == 5))
            def _():
                pltpu.async_copy(f3_hbm.at[sl, pl.ds(0, 128)], dstB, sem_d)

            @pl.when((uu == 6) | (uu == 7))
            def _():
                pltpu.async_copy(f3_hbm.at[sl, pl.ds(128, 128)], dstB, sem_d)

        def _wait_dat(npts, datA, datB, sem_d):
            @pl.when(is_wide)
            def _():
                pltpu.make_async_copy(
                    f2_hbm.at[pl.ds(0, npts), pl.ds(0, 128)],
                    datB.at[pl.ds(0, npts)], sem_d).wait()

            @pl.when(jnp.logical_not(is_wide))
            def _():
                pltpu.make_async_copy(
                    f0_hbm.at[pl.ds(0, npts)],
                    datA.at[pl.ds(0, npts)], sem_d).wait()

        def _compute_idx(ngroups):
            def _idx(i, carry2):
                v = pl.ds(i * 16, 16)
                ids = btv[v] * MAX_SEG + psv[v]
                idxv[v] = ids * 64

                @pl.when(uu == 0)
                def _():
                    plsc.addupdate_scatter(cnt, [ids], ones16)
                return carry2
            lax.fori_loop(0, ngroups, _idx, 0)

        def _acc_group(datv, coff, base, lo, ids2):
            for jj in range(lo, 16):
                rowoff = ids2[jj]
                p = base + jj
                for kk in range(4):
                    plsc.addupdate(acc.at[pl.ds(rowoff + kk * 16, 16)],
                                   datv[p, pl.ds(coff + kk * 16, 16)])

        def _accumulate(datv, coff):
            @plsc.parallel_loop(0, NVEC, unroll=2)
            def _grp(i):
                base = i * 16
                ids2 = idxv[pl.ds(base, 16)]
                _acc_group(datv, coff, base, 0, ids2)

        def _half(j, datA, datB, sem_d, datAn, datBn, semn):
            _wait_idx(CH)
            _compute_idx(NVEC)

            @pl.when(j + 1 < NCHUNK)
            def _():
                _issue_idx(qbase + (j + 1) * CH, CH)
            _wait_dat(CH, datA, datB, sem_d)

            @pl.when(j + 1 < NCHUNK)
            def _():
                _issue_dat(qbase + (j + 1) * CH, CH, datAn, datBn, semn)

            @pl.when(is_wide)
            def _():
                _accumulate(datB, coff_s)

            @pl.when(jnp.logical_not(is_wide))
            def _():
                _accumulate(datA, 0)

        _issue_idx(qbase, CH)
        _issue_dat(qbase, CH, datA0, datB0, sem_d0)

        def _pair(p2, carry):
            jA = 2 * p2
            _half(jA, datA0, datB0, sem_d0, datA1, datB1, sem_d1)

            @pl.when(jA + 1 < NCHUNK)
            def _():
                _half(jA + 1, datA1, datB1, sem_d1, datA0, datB0, sem_d0)
            return carry
        lax.fori_loop(0, (NCHUNK + 1) // 2, _pair, 0)

        tstart = qbase + NCHUNK * CH
        _issue_idx(tstart, TAILP)
        _issue_dat(tstart, TAILP, datA0, datB0, sem_d0)
        _wait_idx(TAILP)
        _compute_idx(TAILP // 16)

        v = pl.ds(TAILP - 16, 16)
        ids = btv[v] * MAX_SEG + psv[v]
        idxv[v] = ids * 64

        @pl.when(uu == 0)
        def _():
            plsc.addupdate_scatter(cnt, [ids], ones16, mask=tail_mask)

        _wait_dat(TAILP, datA0, datB0, sem_d0)

        def _acc_tail(datv, coff):
            for i in range(TAILP // 16):
                ids2 = idxv[pl.ds(i * 16, 16)]
                _acc_group(datv, coff, i * 16, 0, ids2)
            ids3 = idxv[pl.ds(TAILP - 16, 16)]
            _acc_group(datv, coff, TAILP - 16, 8, ids3)

        @pl.when(is_wide)
        def _():
            _acc_tail(datB0, coff_s)

        @pl.when(jnp.logical_not(is_wide))
        def _():
            _acc_tail(datA0, 0)

        pltpu.sync_copy(acc, out_hbm.at[pl.ds(wid * (NSEG * 64), NSEG * 64)])

        @pl.when(uu == 0)
        def _():
            pltpu.sync_copy(cnt, cnt_hbm.at[pl.ds(qq * NSEG, NSEG)])

    return k


_UNITS = [(0, 0), (1, 0), (2, 0), (2, 64), (3, 0), (3, 64), (3, 128), (3, 192)]


def _tc_body(pp, pc,
             W0, b0, g0, be0, W1, b1, g1, be1,
             W2, b2, g2, be2, W3, b3, g3, be3,
             o0, o1, o2, o3):
    cnt = pc[0] + pc[1] + pc[2] + pc[3]
    den = jnp.maximum(cnt, 1.0)[:, None]
    Ws = [W0, W1, W2, W3]
    bs = [b0, b1, b2, b3]
    gs = [g0, g1, g2, g3]
    bes = [be0, be1, be2, be3]
    os_ = [o0, o1, o2, o3]

    projs = [None] * 4
    for u, (lvl, off) in enumerate(_UNITS):
        s = pp[0, u] + pp[1, u] + pp[2, u] + pp[3, u]
        m = s / den
        part = jnp.dot(m, Ws[lvl][pl.ds(off, 64), :],
                       preferred_element_type=_f32)
        projs[lvl] = part if projs[lvl] is None else projs[lvl] + part

    for lvl in range(4):
        x = projs[lvl] + bs[lvl][...]
        mu = jnp.mean(x, axis=-1, keepdims=True)
        var = jnp.mean((x - mu) ** 2, axis=-1, keepdims=True)
        os_[lvl][...] = (x - mu) * lax.rsqrt(var + 1e-5) * gs[lvl][...] + bes[lvl][...]


def kernel(feat0, feat1, feat2, feat3, batch, point2segment, max_seg,
           W0, b0, g0, beta0, W1, b1, g1, beta1,
           W2, b2, g2, beta2, W3, b3, g3, beta3):
    sck = _make_sc_scatter()
    pp, pc = sck(batch, point2segment, feat0, feat1, feat2, feat3)
    pp = pp.reshape(NQ, NUNIT, NSEG, 64)
    pc = pc.reshape(NQ, NSEG)
    o0, o1, o2, o3 = pl.pallas_call(
        _tc_body,
        out_shape=[jax.ShapeDtypeStruct((NSEG, HIDDEN), _f32)] * 4,
    )(pp, pc,
      W0, b0, g0, beta0, W1, b1, g1, beta1,
      W2, b2, g2, beta2, W3, b3, g3, beta3)
    outs = [o.reshape(BATCH, MAX_SEG, HIDDEN) for o in (o0, o1, o2, o3)]
    return (outs[3], outs[2], outs[1], outs[0])


# speedup vs baseline: 1.1879x; 1.1879x over previous
import functools

import jax
import jax.numpy as jnp
from jax import lax
from jax.experimental import pallas as pl
from jax.experimental.pallas import tpu as pltpu
from jax.experimental.pallas import tpu_sc as plsc

N = 100000
BATCH = 4
MAX_SEG = 400
NSEG = BATCH * MAX_SEG
HIDDEN = 256
NQ = 4
QPTS = N // NQ
CH = 200
NCHUNK = QPTS // CH
NVEC = CH // 16
TAILV = CH - NVEC * 16
NUNIT = 8

_f32 = jnp.float32


def _make_sc_scatter():
    mesh = plsc.VectorSubcoreMesh(core_axis_name="c", subcore_axis_name="s")
    out_type = [
        jax.ShapeDtypeStruct((NUNIT, NQ, NSEG * 64), _f32),
        jax.ShapeDtypeStruct((NQ, NSEG), _f32),
    ]
    scratch_types = [
        pltpu.VMEM((NSEG * 64,), _f32),
        pltpu.VMEM((NSEG,), _f32),
        pltpu.VMEM((CH,), jnp.int32),
        pltpu.VMEM((CH,), jnp.int32),
        pltpu.VMEM((CH,), jnp.int32),
        pltpu.VMEM((CH, 64), _f32),
        pltpu.VMEM((CH, 64), _f32),
        pltpu.SemaphoreType.DMA,
        pltpu.SemaphoreType.DMA,
        pltpu.SemaphoreType.DMA,
    ]

    @functools.partial(
        pl.kernel, mesh=mesh, out_type=out_type, scratch_types=scratch_types,
        compiler_params=pltpu.CompilerParams(use_tc_tiling_on_sc=False,
                                             needs_layout_passes=False))
    def k(bat_hbm, seg_hbm, f0_hbm, f1_hbm, f2_hbm, f3_hbm,
          out_hbm, cnt_hbm,
          acc, cnt, btv, psv, idxv, datv0, datv1, sem_i, sem_d0, sem_d1):
        cc = lax.axis_index("c")
        ss = lax.axis_index("s")
        wid = ss * 2 + cc
        uu = wid % NUNIT
        qq = wid // NUNIT
        qbase = qq * QPTS

        @plsc.parallel_loop(0, NSEG, unroll=2)
        def _z(i):
            for kk in range(4):
                acc[pl.ds(i * 64 + kk * 16, 16)] = jnp.zeros((16,), _f32)

        @plsc.parallel_loop(0, NSEG // 16, unroll=2)
        def _zc(i):
            cnt[pl.ds(i * 16, 16)] = jnp.zeros((16,), _f32)

        ones16 = jnp.ones((16,), _f32)
        tail_mask = jax.lax.iota(jnp.int32, 16) >= (16 - TAILV)

        def _issue_idx(j):
            sl = pl.ds(qbase + j * CH, CH)
            pltpu.async_copy(bat_hbm.at[sl], btv, sem_i)
            pltpu.async_copy(seg_hbm.at[sl], psv, sem_i)

        def _wait_idx():
            pltpu.make_async_copy(bat_hbm.at[pl.ds(0, CH)], btv, sem_i).wait()
            pltpu.make_async_copy(seg_hbm.at[pl.ds(0, CH)], psv, sem_i).wait()

        def _issue_dat(j, datv, sem_d):
            sl = pl.ds(qbase + j * CH, CH)

            @pl.when(uu == 0)
            def _():
                pltpu.async_copy(f0_hbm.at[sl], datv, sem_d)

            @pl.when(uu == 1)
            def _():
                pltpu.async_copy(f1_hbm.at[sl], datv, sem_d)

            @pl.when(uu == 2)
            def _():
                pltpu.async_copy(f2_hbm.at[sl, pl.ds(0, 64)], datv, sem_d)

            @pl.when(uu == 3)
            def _():
                pltpu.async_copy(f2_hbm.at[sl, pl.ds(64, 64)], datv, sem_d)

            @pl.when(uu == 4)
            def _():
                pltpu.async_copy(f3_hbm.at[sl, pl.ds(0, 64)], datv, sem_d)

            @pl.when(uu == 5)
            def _():
                pltpu.async_copy(f3_hbm.at[sl, pl.ds(64, 64)], datv, sem_d)

            @pl.when(uu == 6)
            def _():
                pltpu.async_copy(f3_hbm.at[sl, pl.ds(128, 64)], datv, sem_d)

            @pl.when(uu == 7)
            def _():
                pltpu.async_copy(f3_hbm.at[sl, pl.ds(192, 64)], datv, sem_d)

        def _wait_dat(datv, sem_d):
            pltpu.make_async_copy(
                f0_hbm.at[pl.ds(0, CH)], datv, sem_d).wait()

        def _compute_idx():
            @plsc.parallel_loop(0, NVEC, unroll=2)
            def _idx(i):
                v = pl.ds(i * 16, 16)
                ids = btv[v] * MAX_SEG + psv[v]
                idxv[v] = ids * 64

                @pl.when(uu == 0)
                def _():
                    plsc.addupdate_scatter(cnt, [ids], ones16)

            v = pl.ds(CH - 16, 16)
            ids = btv[v] * MAX_SEG + psv[v]
            idxv[v] = ids * 64

            @pl.when(uu == 0)
            def _():
                plsc.addupdate_scatter(cnt, [ids], ones16, mask=tail_mask)

        def _accumulate(datv):
            @plsc.parallel_loop(0, NVEC, unroll=4)
            def _grp(i):
                base = i * 16
                ids2 = idxv[pl.ds(base, 16)]
                for jj in range(16):
                    rowoff = ids2[jj]
                    p = base + jj
                    for kk in range(4):
                        plsc.addupdate(acc.at[pl.ds(rowoff + kk * 16, 16)],
                                       datv[p, pl.ds(kk * 16, 16)])

            ids3 = idxv[pl.ds(CH - 16, 16)]
            for jj in range(16 - TAILV, 16):
                rowoff = ids3[jj]
                p = CH - 16 + jj
                for kk in range(4):
                    plsc.addupdate(acc.at[pl.ds(rowoff + kk * 16, 16)],
                                   datv[p, pl.ds(kk * 16, 16)])

        def _half(j, datv, sem_d, datn, semn):
            _wait_idx()
            _compute_idx()

            @pl.when(j + 1 < NCHUNK)
            def _():
                _issue_idx(j + 1)
            _wait_dat(datv, sem_d)

            @pl.when(j + 1 < NCHUNK)
            def _():
                _issue_dat(j + 1, datn, semn)
            _accumulate(datv)

        _issue_idx(0)
        _issue_dat(0, datv0, sem_d0)

        def _pair(p2, carry):
            jA = 2 * p2
            _half(jA, datv0, sem_d0, datv1, sem_d1)

            @pl.when(jA + 1 < NCHUNK)
            def _():
                _half(jA + 1, datv1, sem_d1, datv0, sem_d0)
            return carry
        lax.fori_loop(0, (NCHUNK + 1) // 2, _pair, 0)

        pltpu.sync_copy(acc, out_hbm.at[uu, qq])

        @pl.when(uu == 0)
        def _():
            pltpu.sync_copy(cnt, cnt_hbm.at[qq])

    return k


_UNITS = [(0, 0), (1, 0), (2, 0), (2, 64), (3, 0), (3, 64), (3, 128), (3, 192)]


def _tc_body(pp, pc,
             W0, b0, g0, be0, W1, b1, g1, be1,
             W2, b2, g2, be2, W3, b3, g3, be3,
             o0, o1, o2, o3):
    cnt = pc[0] + pc[1] + pc[2] + pc[3]
    den = jnp.maximum(cnt, 1.0)[:, None]
    Ws = [W0, W1, W2, W3]
    bs = [b0, b1, b2, b3]
    gs = [g0, g1, g2, g3]
    bes = [be0, be1, be2, be3]
    os_ = [o0, o1, o2, o3]

    projs = [None] * 4
    for u, (lvl, off) in enumerate(_UNITS):
        s = pp[u, 0] + pp[u, 1] + pp[u, 2] + pp[u, 3]
        m = s / den
        part = jnp.dot(m, Ws[lvl][pl.ds(off, 64), :],
                       preferred_element_type=_f32)
        projs[lvl] = part if projs[lvl] is None else projs[lvl] + part

    for lvl in range(4):
        x = projs[lvl] + bs[lvl][...]
        mu = jnp.mean(x, axis=-1, keepdims=True)
        var = jnp.mean((x - mu) ** 2, axis=-1, keepdims=True)
        os_[lvl][...] = (x - mu) * lax.rsqrt(var + 1e-5) * gs[lvl][...] + bes[lvl][...]


def kernel(feat0, feat1, feat2, feat3, batch, point2segment, max_seg,
           W0, b0, g0, beta0, W1, b1, g1, beta1,
           W2, b2, g2, beta2, W3, b3, g3, beta3):
    sck = _make_sc_scatter()
    pp, pc = sck(batch, point2segment, feat0, feat1, feat2, feat3)
    pp = pp.reshape(NUNIT, NQ, NSEG, 64)
    o0, o1, o2, o3 = pl.pallas_call(
        _tc_body,
        out_shape=[jax.ShapeDtypeStruct((NSEG, HIDDEN), _f32)] * 4,
    )(pp, pc,
      W0, b0, g0, beta0, W1, b1, g1, beta1,
      W2, b2, g2, beta2, W3, b3, g3, beta3)
    outs = [o.reshape(BATCH, MAX_SEG, HIDDEN) for o in (o0, o1, o2, o3)]
    return (outs[3], outs[2], outs[1], outs[0])
